# baseline (speedup 1.0000x reference)
import functools

import jax
import jax.numpy as jnp
from jax.experimental import pallas as pl
from jax.experimental.pallas import tpu as pltpu


def _ceil_to(x, m):
    return (x + m - 1) // m * m


def _topk_body(K, N, C, x_ref, poly_ref, scale_ref, scores_ref, labels_ref, polys_ref):
    s = jax.nn.sigmoid(x_ref[0])
    q_iota = jax.lax.broadcasted_iota(jnp.int32, (N, C), 0)
    c_iota = jax.lax.broadcasted_iota(jnp.int32, (N, C), 1)
    flat = q_iota * C + c_iota
    KP = _ceil_to(K, 8)
    lane_iota = jax.lax.broadcasted_iota(jnp.int32, (1, KP), 1)
    sub_iota = jax.lax.broadcasted_iota(jnp.int32, (KP, 1), 0)
    BIG = jnp.int32(1 << 30)

    def step(i, carry):
        s, out_v, out_i, out_i_sub = carry
        m = jnp.max(s)
        cand = jnp.where(s == m, flat, BIG)
        fi = jnp.min(cand)
        out_v = jnp.where(lane_iota == i, m, out_v)
        out_i = jnp.where(lane_iota == i, fi, out_i)
        out_i_sub = jnp.where(sub_iota == i, fi, out_i_sub)
        s = jnp.where(flat == fi, -1.0, s)
        return s, out_v, out_i, out_i_sub

    init = (
        s,
        jnp.zeros((1, KP), jnp.float32),
        jnp.zeros((1, KP), jnp.int32),
        jnp.zeros((KP, 1), jnp.int32),
    )
    _, out_v, out_i, out_i_sub = jax.lax.fori_loop(0, K, step, init)

    scores_ref[0, 0, :] = out_v[0, :K]
    labels_ref[0, 0, :] = out_i[0, :K] % C
    q_sub = out_i_sub // C
    oh = (q_sub == jax.lax.broadcasted_iota(jnp.int32, (KP, N), 1)).astype(jnp.float32)
    polys = jax.lax.dot_general(
        oh, poly_ref[0],
        dimension_numbers=(((1,), (0,)), ((), ())),
        preferred_element_type=jnp.float32,
    )
    polys_ref[0] = (polys * scale_ref[0, 0, :])[:K]


def kernel(pred_logits, pred_polys, target_sizes):
    B, N, C = pred_logits.shape
    K = N
    D = pred_polys.shape[-1]
    img_h = target_sizes[:, 0].astype(jnp.float32)
    img_w = target_sizes[:, 1].astype(jnp.float32)
    scale = jnp.stack([img_w, img_h] * (D // 2), axis=1).reshape(B, 1, D)

    grid = (B,)
    out = pl.pallas_call(
        functools.partial(_topk_body, K, N, C),
        grid=grid,
        in_specs=[
            pl.BlockSpec((1, N, C), lambda b: (b, 0, 0)),
            pl.BlockSpec((1, N, D), lambda b: (b, 0, 0)),
            pl.BlockSpec((1, 1, D), lambda b: (b, 0, 0)),
        ],
        out_specs=[
            pl.BlockSpec((1, 1, K), lambda b: (b, 0, 0)),
            pl.BlockSpec((1, 1, K), lambda b: (b, 0, 0)),
            pl.BlockSpec((1, K, D), lambda b: (b, 0, 0)),
        ],
        out_shape=[
            jax.ShapeDtypeStruct((B, 1, K), jnp.float32),
            jax.ShapeDtypeStruct((B, 1, K), jnp.int32),
            jax.ShapeDtypeStruct((B, K, D), jnp.float32),
        ],
    )(pred_logits, pred_polys, scale)
    scores, labels, polys = out
    return scores.reshape(B, K), labels.reshape(B, K), polys


# speedup vs baseline: 2.8762x; 2.8762x over previous
import functools

import jax
import jax.numpy as jnp
from jax.experimental import pallas as pl
from jax.experimental.pallas import tpu as pltpu


def _next_pow2(x):
    p = 1
    while p < x:
        p *= 2
    return p


def _two_word_gt(a_v, b_v, a_i, b_i):
    return (a_v > b_v) | ((a_v == b_v) & (a_i < b_i))


def _cmpex(vals, idxs, stride, size, half_lanes):
    S, W = vals.shape
    G = S // (2 * stride)
    v4 = vals.reshape(G, 2, stride, W)
    i4 = idxs.reshape(G, 2, stride, W)
    a_v, b_v = v4[:, 0], v4[:, 1]
    a_i, b_i = i4[:, 0], i4[:, 1]
    a_win = _two_word_gt(a_v, b_v, a_i, b_i)
    if size is None:
        dir_desc = None
    else:
        g = jax.lax.broadcasted_iota(jnp.int32, (G, stride, W), 0)
        dir_desc = ((g * (2 * stride)) & size) == 0
    if half_lanes is not None:
        lane = jax.lax.broadcasted_iota(jnp.int32, (G, stride, W), 2)
        lane_desc = lane < half_lanes
        dir_desc = lane_desc if dir_desc is None else (dir_desc == lane_desc)
    keep = a_win if dir_desc is None else (a_win == dir_desc)
    new_a_v = jnp.where(keep, a_v, b_v)
    new_b_v = jnp.where(keep, b_v, a_v)
    new_a_i = jnp.where(keep, a_i, b_i)
    new_b_i = jnp.where(keep, b_i, a_i)
    vals = jnp.concatenate([new_a_v[:, None], new_b_v[:, None]], axis=1).reshape(S, W)
    idxs = jnp.concatenate([new_a_i[:, None], new_b_i[:, None]], axis=1).reshape(S, W)
    return vals, idxs


def _topk_body(K, N, C, SP, x_ref, poly_ref, scale_ref, scores_ref, labels_ref, polys_ref):
    s = jax.nn.sigmoid(x_ref[0])
    if SP > N:
        s = jnp.concatenate([s, jnp.full((SP - N, C), -1.0, jnp.float32)], axis=0)
    idx = (jax.lax.broadcasted_iota(jnp.int32, (SP, C), 0) * C
           + jax.lax.broadcasted_iota(jnp.int32, (SP, C), 1))

    size = 2
    while size <= SP:
        stride = size // 2
        while stride >= 1:
            s, idx = _cmpex(s, idx, stride, size, C // 2 if C > 1 else None)
            stride //= 2
        size *= 2

    W = C
    while W > 1:
        W //= 2
        a_v, b_v = s[:, :W], s[:, W:2 * W]
        a_i, b_i = idx[:, :W], idx[:, W:2 * W]
        win = _two_word_gt(a_v, b_v, a_i, b_i)
        s = jnp.where(win, a_v, b_v)
        idx = jnp.where(win, a_i, b_i)
        half = W // 2 if W > 1 else None
        stride = SP // 2
        while stride >= 1:
            s, idx = _cmpex(s, idx, stride, None, half)
            stride //= 2

    s_lane = s.reshape(1, SP)
    i_lane = idx.reshape(1, SP)
    scores_ref[0, 0, :] = s_lane[0, :K]
    labels_ref[0, 0, :] = i_lane[0, :K] % C

    q_col = idx // C
    oh = (q_col == jax.lax.broadcasted_iota(jnp.int32, (SP, N), 1)).astype(jnp.float32)
    polys = jax.lax.dot_general(
        oh, poly_ref[0],
        dimension_numbers=(((1,), (0,)), ((), ())),
        preferred_element_type=jnp.float32,
    )
    polys_ref[0] = (polys * scale_ref[0, 0, :])[:K]


def kernel(pred_logits, pred_polys, target_sizes):
    B, N, C = pred_logits.shape
    K = N
    D = pred_polys.shape[-1]
    SP = _next_pow2(N)
    img_h = target_sizes[:, 0].astype(jnp.float32)
    img_w = target_sizes[:, 1].astype(jnp.float32)
    scale = jnp.stack([img_w, img_h] * (D // 2), axis=1).reshape(B, 1, D)

    out = pl.pallas_call(
        functools.partial(_topk_body, K, N, C, SP),
        grid=(B,),
        in_specs=[
            pl.BlockSpec((1, N, C), lambda b: (b, 0, 0)),
            pl.BlockSpec((1, N, D), lambda b: (b, 0, 0)),
            pl.BlockSpec((1, 1, D), lambda b: (b, 0, 0)),
        ],
        out_specs=[
            pl.BlockSpec((1, 1, K), lambda b: (b, 0, 0)),
            pl.BlockSpec((1, 1, K), lambda b: (b, 0, 0)),
            pl.BlockSpec((1, K, D), lambda b: (b, 0, 0)),
        ],
        out_shape=[
            jax.ShapeDtypeStruct((B, 1, K), jnp.float32),
            jax.ShapeDtypeStruct((B, 1, K), jnp.int32),
            jax.ShapeDtypeStruct((B, K, D), jnp.float32),
        ],
    )(pred_logits, pred_polys, scale)
    scores, labels, polys = out
    return scores.reshape(B, K), labels.reshape(B, K), polys


# speedup vs baseline: 10.0718x; 3.5018x over previous
import functools

import jax
import jax.numpy as jnp
from jax.experimental import pallas as pl
from jax.experimental.pallas import tpu as pltpu


def _next_pow2(x):
    p = 1
    while p < x:
        p *= 2
    return p


def _two_word_gt(a_v, b_v, a_i, b_i):
    return (a_v > b_v) | ((a_v == b_v) & (a_i < b_i))


def _cmpex_roll(vals, idxs, stride, size, half_lanes):
    S, W = vals.shape
    sub = jax.lax.broadcasted_iota(jnp.int32, (S, W), 0)
    first = (sub & stride) == 0
    p_v = jnp.where(first, pltpu.roll(vals, S - stride, 0), pltpu.roll(vals, stride, 0))
    p_i = jnp.where(first, pltpu.roll(idxs, S - stride, 0), pltpu.roll(idxs, stride, 0))
    i_win = _two_word_gt(vals, p_v, idxs, p_i)
    dir_desc = None if size is None else ((sub & size) == 0)
    if half_lanes is not None:
        lane_desc = jax.lax.broadcasted_iota(jnp.int32, (S, W), 1) < half_lanes
        dir_desc = lane_desc if dir_desc is None else (dir_desc == lane_desc)
    keep_winner = first if dir_desc is None else (first == dir_desc)
    new_v = jnp.where(i_win == keep_winner, vals, p_v)
    new_i = jnp.where(i_win == keep_winner, idxs, p_i)
    return new_v, new_i


def _cmpex(vals, idxs, stride, size, half_lanes):
    if stride < 8:
        return _cmpex_roll(vals, idxs, stride, size, half_lanes)
    S, W = vals.shape
    G = S // (2 * stride)
    v4 = vals.reshape(G, 2, stride, W)
    i4 = idxs.reshape(G, 2, stride, W)
    a_v, b_v = v4[:, 0], v4[:, 1]
    a_i, b_i = i4[:, 0], i4[:, 1]
    a_win = _two_word_gt(a_v, b_v, a_i, b_i)
    if size is None:
        dir_desc = None
    else:
        g = jax.lax.broadcasted_iota(jnp.int32, (G, stride, W), 0)
        dir_desc = ((g * (2 * stride)) & size) == 0
    if half_lanes is not None:
        lane = jax.lax.broadcasted_iota(jnp.int32, (G, stride, W), 2)
        lane_desc = lane < half_lanes
        dir_desc = lane_desc if dir_desc is None else (dir_desc == lane_desc)
    keep = a_win if dir_desc is None else (a_win == dir_desc)
    new_a_v = jnp.where(keep, a_v, b_v)
    new_b_v = jnp.where(keep, b_v, a_v)
    new_a_i = jnp.where(keep, a_i, b_i)
    new_b_i = jnp.where(keep, b_i, a_i)
    vals = jnp.concatenate([new_a_v[:, None], new_b_v[:, None]], axis=1).reshape(S, W)
    idxs = jnp.concatenate([new_a_i[:, None], new_b_i[:, None]], axis=1).reshape(S, W)
    return vals, idxs


def _topk_body(K, N, C, SP, x_ref, poly_ref, scale_ref, scores_ref, labels_ref, polys_ref):
    s = jax.nn.sigmoid(x_ref[0])
    if SP > N:
        s = jnp.concatenate([s, jnp.full((SP - N, C), -1.0, jnp.float32)], axis=0)
    idx = (jax.lax.broadcasted_iota(jnp.int32, (SP, C), 0) * C
           + jax.lax.broadcasted_iota(jnp.int32, (SP, C), 1))

    size = 2
    while size <= SP:
        stride = size // 2
        while stride >= 1:
            s, idx = _cmpex(s, idx, stride, size, C // 2 if C > 1 else None)
            stride //= 2
        size *= 2

    W = C
    while W > 1:
        W //= 2
        a_v, b_v = s[:, :W], s[:, W:2 * W]
        a_i, b_i = idx[:, :W], idx[:, W:2 * W]
        win = _two_word_gt(a_v, b_v, a_i, b_i)
        s = jnp.where(win, a_v, b_v)
        idx = jnp.where(win, a_i, b_i)
        half = W // 2 if W > 1 else None
        stride = SP // 2
        while stride >= 1:
            s, idx = _cmpex(s, idx, stride, None, half)
            stride //= 2

    s_lane = s.reshape(1, SP)
    i_lane = idx.reshape(1, SP)
    scores_ref[0, 0, :] = s_lane[0, :K]
    labels_ref[0, 0, :] = i_lane[0, :K] % C

    q_col = idx // C
    oh = (q_col == jax.lax.broadcasted_iota(jnp.int32, (SP, N), 1)).astype(jnp.float32)
    polys = jax.lax.dot_general(
        oh, poly_ref[0],
        dimension_numbers=(((1,), (0,)), ((), ())),
        preferred_element_type=jnp.float32,
    )
    polys_ref[0] = (polys * scale_ref[0, 0, :])[:K]


def kernel(pred_logits, pred_polys, target_sizes):
    B, N, C = pred_logits.shape
    K = N
    D = pred_polys.shape[-1]
    SP = _next_pow2(N)
    img_h = target_sizes[:, 0].astype(jnp.float32)
    img_w = target_sizes[:, 1].astype(jnp.float32)
    scale = jnp.stack([img_w, img_h] * (D // 2), axis=1).reshape(B, 1, D)

    out = pl.pallas_call(
        functools.partial(_topk_body, K, N, C, SP),
        grid=(B,),
        in_specs=[
            pl.BlockSpec((1, N, C), lambda b: (b, 0, 0)),
            pl.BlockSpec((1, N, D), lambda b: (b, 0, 0)),
            pl.BlockSpec((1, 1, D), lambda b: (b, 0, 0)),
        ],
        out_specs=[
            pl.BlockSpec((1, 1, K), lambda b: (b, 0, 0)),
            pl.BlockSpec((1, 1, K), lambda b: (b, 0, 0)),
            pl.BlockSpec((1, K, D), lambda b: (b, 0, 0)),
        ],
        out_shape=[
            jax.ShapeDtypeStruct((B, 1, K), jnp.float32),
            jax.ShapeDtypeStruct((B, 1, K), jnp.int32),
            jax.ShapeDtypeStruct((B, K, D), jnp.float32),
        ],
    )(pred_logits, pred_polys, scale)
    scores, labels, polys = out
    return scores.reshape(B, K), labels.reshape(B, K), polys
